# baseline (speedup 1.0000x reference)
import functools

import jax
import jax.numpy as jnp
from jax import lax
from jax.experimental import pallas as pl
from jax.experimental.pallas import tpu as pltpu
from jax.experimental.pallas import tpu_sc as plsc

BB = 8
QL = 14
KVG = 36
KQG = 14
NWORD = 10
NBR = 4
EMB = 300
HID = 1024
DVG = 2048
DGM = 2048
OUTD = 3129
SEQ = BB + BB * KQG
EPAD = 384
NIDS = 2 * SEQ * QL
NIDS_PAD = 3584
PER_TILE = NIDS_PAD // 32


def _sc_gather(table_pad, ids):
    mesh = plsc.VectorSubcoreMesh(core_axis_name="c", subcore_axis_name="s")

    @functools.partial(
        pl.kernel,
        mesh=mesh,
        out_type=jax.ShapeDtypeStruct((NIDS_PAD, EPAD), jnp.float32),
        scratch_types=[
            pltpu.VMEM((PER_TILE,), jnp.int32),
            pltpu.VMEM((PER_TILE, EPAD), jnp.float32),
            pltpu.SemaphoreType.DMA,
        ],
    )
    def gk(table_hbm, idx_hbm, out_hbm, idx_v, rows_v, sem):
        wid = lax.axis_index("s") * 2 + lax.axis_index("c")
        base = wid * PER_TILE
        pltpu.sync_copy(idx_hbm.at[pl.ds(base, PER_TILE)], idx_v)
        pltpu.async_copy(table_hbm.at[idx_v], rows_v, sem).wait()
        pltpu.sync_copy(rows_v, out_hbm.at[pl.ds(base, PER_TILE)])

    return gk(table_pad, ids)


def _pad_body(x_ref, o_ref):
    o_ref[...] = jnp.concatenate(
        [x_ref[...], jnp.zeros((x_ref.shape[0], EPAD - EMB), jnp.float32)],
        axis=1)


def _pad_table(table):
    rows = table.shape[0]
    rt = 2000
    return pl.pallas_call(
        _pad_body,
        grid=(rows // rt,),
        in_specs=[pl.BlockSpec((rt, EMB), lambda r: (r, 0))],
        out_specs=pl.BlockSpec((rt, EPAD), lambda r: (r, 0)),
        out_shape=jax.ShapeDtypeStruct((rows, EPAD), jnp.float32),
    )(table)


def _gru_body(xf_ref, xr_ref, lens_ref, wf_ref, uf_ref, bif_ref, bhf_ref,
              wb_ref, ub_ref, bib_ref, bhb_ref, out_ref, hf_ref, hb_ref,
              wf16_ref, uf16_ref, wb16_ref, ub16_ref):
    t = pl.program_id(0)

    @pl.when(t == 0)
    def _():
        hf_ref[...] = jnp.zeros_like(hf_ref)
        hb_ref[...] = jnp.zeros_like(hb_ref)
        wf16_ref[...] = wf_ref[...].astype(jnp.bfloat16)
        uf16_ref[...] = uf_ref[...].astype(jnp.bfloat16)
        wb16_ref[...] = wb_ref[...].astype(jnp.bfloat16)
        ub16_ref[...] = ub_ref[...].astype(jnp.bfloat16)

    mask = lens_ref[...] > t

    def step(x_ref, w_ref, u_ref, bi_ref, bh_ref, h_ref):
        x = x_ref[:, :EMB].astype(jnp.bfloat16)
        h = h_ref[...]
        h16 = h.astype(jnp.bfloat16)
        gi = jnp.dot(x, w_ref[...], preferred_element_type=jnp.float32) + bi_ref[...]
        gh = jnp.dot(h16, u_ref[...], preferred_element_type=jnp.float32) + bh_ref[...]
        r = jax.nn.sigmoid(gi[:, :HID] + gh[:, :HID])
        z = jax.nn.sigmoid(gi[:, HID:2 * HID] + gh[:, HID:2 * HID])
        n = jnp.tanh(gi[:, 2 * HID:] + r * gh[:, 2 * HID:])
        h_new = (1.0 - z) * n + z * h
        h_ref[...] = jnp.where(mask, h_new, h)

    step(xf_ref, wf16_ref, uf16_ref, bif_ref, bhf_ref, hf_ref)
    step(xr_ref, wb16_ref, ub16_ref, bib_ref, bhb_ref, hb_ref)

    @pl.when(t == QL - 1)
    def _():
        out_ref[:, :HID] = hf_ref[...]
        out_ref[:, HID:] = hb_ref[...]


def _gru_call(G, lens, Wf, Uf, bif, bhf, Wb, Ub, bib, bhb):
    def c2(shape):
        return pl.BlockSpec(shape, lambda t: (0, 0))

    return pl.pallas_call(
        _gru_body,
        grid=(QL,),
        in_specs=[
            pl.BlockSpec((SEQ, EPAD), lambda t: (t, 0)),
            pl.BlockSpec((SEQ, EPAD), lambda t: (t + QL, 0)),
            c2((SEQ, 1)),
            c2((EMB, 3 * HID)), c2((HID, 3 * HID)),
            c2((1, 3 * HID)), c2((1, 3 * HID)),
            c2((EMB, 3 * HID)), c2((HID, 3 * HID)),
            c2((1, 3 * HID)), c2((1, 3 * HID)),
        ],
        out_specs=pl.BlockSpec((SEQ, 2 * HID), lambda t: (0, 0)),
        out_shape=jax.ShapeDtypeStruct((SEQ, 2 * HID), jnp.float32),
        scratch_shapes=[pltpu.VMEM((SEQ, HID), jnp.float32),
                        pltpu.VMEM((SEQ, HID), jnp.float32),
                        pltpu.VMEM((EMB, 3 * HID), jnp.bfloat16),
                        pltpu.VMEM((HID, 3 * HID), jnp.bfloat16),
                        pltpu.VMEM((EMB, 3 * HID), jnp.bfloat16),
                        pltpu.VMEM((HID, 3 * HID), jnp.bfloat16)],
    )(G, G, lens, Wf, Uf, bif, bhf, Wb, Ub, bib, bhb)


def _adj_body(ge1_ref, ge2_ref, vgn_ref, qgn_ref,
              a1_ref, a2_ref, m12_ref, m21_ref):
    n1 = BB * KVG
    col1 = lax.broadcasted_iota(jnp.int32, (n1, n1), 1)
    acc1 = jnp.zeros((n1, n1), jnp.float32)
    for k in range(NBR):
        acc1 = acc1 + (ge1_ref[:, k:k + 1] == col1).astype(jnp.float32)
    a1_ref[...] = acc1 * (1.0 / NBR)

    n2 = BB * KQG
    col2 = lax.broadcasted_iota(jnp.int32, (n2, n2), 1)
    acc2 = jnp.zeros((n2, n2), jnp.float32)
    for k in range(NBR):
        acc2 = acc2 + (ge2_ref[:, k:k + 1] == col2).astype(jnp.float32)
    a2_ref[...] = acc2 * (1.0 / NBR)

    vs0 = jnp.sum(jnp.abs(vgn_ref[...]), axis=2) == 0.0
    qs0 = jnp.sum(jnp.abs(qgn_ref[...]), axis=2) == 0
    neg = jnp.float32(-1e9)
    rows12, rows21 = [], []
    for b in range(BB):
        mid12 = jnp.where(jnp.broadcast_to(qs0[b:b + 1, :], (KVG, KQG)), neg, 0.0)
        parts = []
        if b > 0:
            parts.append(jnp.full((KVG, KQG * b), neg, jnp.float32))
        parts.append(mid12)
        if b < BB - 1:
            parts.append(jnp.full((KVG, KQG * (BB - 1 - b)), neg, jnp.float32))
        rows12.append(jnp.concatenate(parts, axis=1) if len(parts) > 1 else parts[0])

        mid21 = jnp.where(jnp.broadcast_to(vs0[b:b + 1, :], (KQG, KVG)), neg, 0.0)
        parts = []
        if b > 0:
            parts.append(jnp.full((KQG, KVG * b), neg, jnp.float32))
        parts.append(mid21)
        if b < BB - 1:
            parts.append(jnp.full((KQG, KVG * (BB - 1 - b)), neg, jnp.float32))
        rows21.append(jnp.concatenate(parts, axis=1) if len(parts) > 1 else parts[0])
    m12_ref[...] = jnp.concatenate(rows12, axis=0)
    m21_ref[...] = jnp.concatenate(rows21, axis=0)


def _adj_call(ge1, ge2, vg_nodes, qg_nodes):
    return pl.pallas_call(
        _adj_body,
        out_shape=[
            jax.ShapeDtypeStruct((BB * KVG, BB * KVG), jnp.float32),
            jax.ShapeDtypeStruct((BB * KQG, BB * KQG), jnp.float32),
            jax.ShapeDtypeStruct((BB * KVG, BB * KQG), jnp.float32),
            jax.ShapeDtypeStruct((BB * KQG, BB * KVG), jnp.float32),
        ],
    )(ge1, ge2, vg_nodes, qg_nodes)


def _softmax_last(x):
    mx = jnp.max(x, axis=-1, keepdims=True)
    e = jnp.exp(x - mx)
    return e / jnp.sum(e, axis=-1, keepdims=True)


BNL = 256
NCT = DGM // BNL


def _layer_body(x1_ref, x2_ref, a1_ref, a2_ref, m12_ref, m21_ref,
                ws1_ref, wn1_ref, ws2_ref, wn2_ref, o1_ref, o2_ref,
                h1s_ref, h2s_ref, s12_ref, s21_ref):
    p = pl.program_id(0)
    c = pl.program_id(1)
    dn = (((1,), (1,)), ((), ()))

    @pl.when(p == 0)
    def _():
        x1 = x1_ref[...]
        x2 = x2_ref[...]
        z1 = jnp.dot(x1, wn1_ref[...], preferred_element_type=jnp.float32)
        h1c = jnp.maximum(
            jnp.dot(x1, ws1_ref[...], preferred_element_type=jnp.float32)
            + jnp.dot(a1_ref[...], z1, preferred_element_type=jnp.float32), 0.0)
        z2 = jnp.dot(x2, wn2_ref[...], preferred_element_type=jnp.float32)
        h2c = jnp.maximum(
            jnp.dot(x2, ws2_ref[...], preferred_element_type=jnp.float32)
            + jnp.dot(a2_ref[...], z2, preferred_element_type=jnp.float32), 0.0)
        h1s_ref[c] = h1c
        h2s_ref[c] = h2c
        ps12 = lax.dot_general(h1c, h2c, dn, preferred_element_type=jnp.float32)
        ps21 = lax.dot_general(h2c, h1c, dn, preferred_element_type=jnp.float32)

        @pl.when(c == 0)
        def _():
            s12_ref[...] = ps12
            s21_ref[...] = ps21

        @pl.when(c > 0)
        def _():
            s12_ref[...] += ps12
            s21_ref[...] += ps21

        @pl.when(c == NCT - 1)
        def _():
            scale = 1.0 / (DGM ** 0.5)
            s12_ref[...] = _softmax_last(s12_ref[...] * scale + m12_ref[...])
            s21_ref[...] = _softmax_last(s21_ref[...] * scale + m21_ref[...])

    @pl.when(p == 1)
    def _():
        h1c = h1s_ref[c]
        h2c = h2s_ref[c]
        o1_ref[...] = h1c + jnp.dot(s12_ref[...], h2c,
                                    preferred_element_type=jnp.float32)
        o2_ref[...] = h2c + jnp.dot(s21_ref[...], h1c,
                                    preferred_element_type=jnp.float32)


def _layer(x1, x2, a1, a2, m12, m21, ws1, wn1, ws2, wn2):
    m1, d = x1.shape
    m2 = x2.shape[0]

    def const(shape):
        return pl.BlockSpec(shape, lambda p, c: (0, 0))

    def wspec(shape):
        return pl.BlockSpec(shape, lambda p, c: (0, c * (1 - p)))

    return pl.pallas_call(
        _layer_body,
        grid=(2, NCT),
        in_specs=[
            const((m1, d)), const((m2, d)),
            const((m1, m1)), const((m2, m2)),
            const((m1, m2)), const((m2, m1)),
            wspec((d, BNL)), wspec((d, BNL)),
            wspec((d, BNL)), wspec((d, BNL)),
        ],
        out_specs=[
            pl.BlockSpec((m1, BNL), lambda p, c: (0, c)),
            pl.BlockSpec((m2, BNL), lambda p, c: (0, c)),
        ],
        out_shape=[
            jax.ShapeDtypeStruct((m1, DGM), jnp.float32),
            jax.ShapeDtypeStruct((m2, DGM), jnp.float32),
        ],
        scratch_shapes=[
            pltpu.VMEM((NCT, m1, BNL), jnp.float32),
            pltpu.VMEM((NCT, m2, BNL), jnp.float32),
            pltpu.VMEM((m1, m2), jnp.float32),
            pltpu.VMEM((m2, m1), jnp.float32),
        ],
    )(x1, x2, a1, a2, m12, m21, ws1, wn1, ws2, wn2)


def _head1_body(x2_ref, q_ref, w1_ref, b1_ref, o_ref):
    ffs = [jnp.max(x2_ref[b * KQG:(b + 1) * KQG, :], axis=0, keepdims=True)
           for b in range(BB)]
    ff = jnp.concatenate(ffs, axis=0)
    h = jnp.maximum(q_ref[...], 0.0) * ff
    o_ref[...] = jnp.maximum(
        jnp.dot(h, w1_ref[...], preferred_element_type=jnp.float32) + b1_ref[...],
        0.0)


def _head1(x2, qenc, w1, b1):
    bn = 512
    g = (OUTD + bn - 1) // bn
    return pl.pallas_call(
        _head1_body,
        grid=(g,),
        in_specs=[
            pl.BlockSpec((BB * KQG, DGM), lambda c: (0, 0)),
            pl.BlockSpec((BB, DGM), lambda c: (0, 0)),
            pl.BlockSpec((DGM, bn), lambda c: (0, c)),
            pl.BlockSpec((1, bn), lambda c: (0, c)),
        ],
        out_specs=pl.BlockSpec((BB, bn), lambda c: (0, c)),
        out_shape=jax.ShapeDtypeStruct((BB, OUTD), jnp.float32),
    )(x2, qenc, w1, b1)


def _head2_body(h_ref, w2_ref, b2_ref, o_ref):
    o_ref[...] = jnp.dot(h_ref[...], w2_ref[...],
                         preferred_element_type=jnp.float32) + b2_ref[...]


def _head2(hid1, w2, b2):
    bn = 512
    g = (OUTD + bn - 1) // bn
    return pl.pallas_call(
        _head2_body,
        grid=(g,),
        in_specs=[
            pl.BlockSpec((BB, OUTD), lambda c: (0, 0)),
            pl.BlockSpec((OUTD, bn), lambda c: (0, c)),
            pl.BlockSpec((1, bn), lambda c: (0, c)),
        ],
        out_specs=pl.BlockSpec((BB, bn), lambda c: (0, c)),
        out_shape=jax.ShapeDtypeStruct((BB, OUTD), jnp.float32),
    )(hid1, w2, b2)


def kernel(question, vg_nodes, vg_edges, qg_nodes, qg_edges, qglen, qlen,
           emb_table, Wf, Uf, bif, bhf, Wb, Ub, bib, bhb,
           Ws1a, Wn1a, Ws2a, Wn2a, Ws1b, Wn1b, Ws2b, Wn2b, W1, b1, W2, b2):
    tokq = question.astype(jnp.int32)
    tokg = qg_nodes.reshape(BB * KQG, NWORD).astype(jnp.int32)
    tokg = jnp.pad(tokg, ((0, 0), (0, QL - NWORD)))
    tok_f = jnp.concatenate([tokq, tokg], axis=0)
    lens = jnp.concatenate([qlen.astype(jnp.int32),
                            qglen.reshape(-1).astype(jnp.int32)])
    tt = jnp.arange(QL, dtype=jnp.int32)
    pos = jnp.clip(lens[:, None] - 1 - tt[None, :], 0, QL - 1)
    tok_r = jnp.take_along_axis(tok_f, pos, axis=1)
    ids = jnp.concatenate([
        tok_f.T.reshape(-1), tok_r.T.reshape(-1),
        jnp.zeros((NIDS_PAD - NIDS,), jnp.int32)]).astype(jnp.int32)
    tpad = _pad_table(emb_table)
    G = _sc_gather(tpad, ids)

    H = _gru_call(G, lens.reshape(SEQ, 1),
                  Wf, Uf, bif.reshape(1, -1), bhf.reshape(1, -1),
                  Wb, Ub, bib.reshape(1, -1), bhb.reshape(1, -1))
    qenc = H[:BB]
    qg_enc = H[BB:]

    roff1 = (jnp.arange(BB * KVG, dtype=jnp.int32) // KVG * KVG)[:, None]
    ge1 = vg_edges.reshape(BB * KVG, NBR).astype(jnp.int32) + roff1
    roff2 = (jnp.arange(BB * KQG, dtype=jnp.int32) // KQG * KQG)[:, None]
    ge2 = qg_edges.reshape(BB * KQG, NBR).astype(jnp.int32) + roff2
    A1, A2, M12, M21 = _adj_call(ge1, ge2, vg_nodes, qg_nodes.astype(jnp.int32))

    qb1 = jnp.broadcast_to(qenc[:, None, :], (BB, KVG, DGM)).reshape(BB * KVG, DGM)
    qb2 = jnp.broadcast_to(qenc[:, None, :], (BB, KQG, DGM)).reshape(BB * KQG, DGM)
    x1 = jnp.concatenate([vg_nodes.reshape(BB * KVG, DVG), qb1], axis=1)
    x2 = jnp.concatenate([qg_enc, qb2], axis=1)

    x1, x2 = _layer(x1, x2, A1, A2, M12, M21, Ws1a, Wn1a, Ws2a, Wn2a)
    x1, x2 = _layer(x1, x2, A1, A2, M12, M21, Ws1b, Wn1b, Ws2b, Wn2b)

    hid1 = _head1(x2, qenc, W1, b1.reshape(1, OUTD))
    return _head2(hid1, W2, b2.reshape(1, OUTD))


# speedup vs baseline: 1.0249x; 1.0249x over previous
import functools

import jax
import jax.numpy as jnp
from jax import lax
from jax.experimental import pallas as pl
from jax.experimental.pallas import tpu as pltpu
from jax.experimental.pallas import tpu_sc as plsc

BB = 8
QL = 14
KVG = 36
KQG = 14
NWORD = 10
NBR = 4
EMB = 300
HID = 1024
DVG = 2048
DGM = 2048
OUTD = 3129
SEQ = BB + BB * KQG
EPAD = 384
NIDS = 2 * SEQ * QL
NIDS_PAD = 3584
PER_TILE = NIDS_PAD // 32


def _sc_gather(table_pad, ids):
    mesh = plsc.VectorSubcoreMesh(core_axis_name="c", subcore_axis_name="s")

    @functools.partial(
        pl.kernel,
        mesh=mesh,
        out_type=jax.ShapeDtypeStruct((NIDS_PAD, EPAD), jnp.float32),
        scratch_types=[
            pltpu.VMEM((PER_TILE,), jnp.int32),
            pltpu.VMEM((PER_TILE, EPAD), jnp.float32),
            pltpu.SemaphoreType.DMA,
        ],
    )
    def gk(table_hbm, idx_hbm, out_hbm, idx_v, rows_v, sem):
        wid = lax.axis_index("s") * 2 + lax.axis_index("c")
        base = wid * PER_TILE
        pltpu.sync_copy(idx_hbm.at[pl.ds(base, PER_TILE)], idx_v)
        pltpu.async_copy(table_hbm.at[idx_v], rows_v, sem).wait()
        pltpu.sync_copy(rows_v, out_hbm.at[pl.ds(base, PER_TILE)])

    return gk(table_pad, ids)


def _pad_body(x_ref, o_ref):
    o_ref[...] = jnp.concatenate(
        [x_ref[...], jnp.zeros((x_ref.shape[0], EPAD - EMB), jnp.float32)],
        axis=1)


def _pad_table(table):
    rows = table.shape[0]
    rt = 2000
    return pl.pallas_call(
        _pad_body,
        grid=(rows // rt,),
        in_specs=[pl.BlockSpec((rt, EMB), lambda r: (r, 0))],
        out_specs=pl.BlockSpec((rt, EPAD), lambda r: (r, 0)),
        out_shape=jax.ShapeDtypeStruct((rows, EPAD), jnp.float32),
    )(table)


def _gru_body(xf_ref, xr_ref, lens_ref, wf_ref, uf_ref, bif_ref, bhf_ref,
              wb_ref, ub_ref, bib_ref, bhb_ref, out_ref, hf_ref, hb_ref,
              wf16_ref, uf16_ref, wb16_ref, ub16_ref):
    t = pl.program_id(0)

    @pl.when(t == 0)
    def _():
        hf_ref[...] = jnp.zeros_like(hf_ref)
        hb_ref[...] = jnp.zeros_like(hb_ref)
        wf16_ref[...] = wf_ref[...].astype(jnp.bfloat16)
        uf16_ref[...] = uf_ref[...].astype(jnp.bfloat16)
        wb16_ref[...] = wb_ref[...].astype(jnp.bfloat16)
        ub16_ref[...] = ub_ref[...].astype(jnp.bfloat16)

    mask = lens_ref[...] > t

    def step(x_ref, w_ref, u_ref, bi_ref, bh_ref, h_ref):
        x = x_ref[:, :EMB].astype(jnp.bfloat16)
        h = h_ref[...]
        h16 = h.astype(jnp.bfloat16)
        gi = jnp.dot(x, w_ref[...], preferred_element_type=jnp.float32) + bi_ref[...]
        gh = jnp.dot(h16, u_ref[...], preferred_element_type=jnp.float32) + bh_ref[...]
        r = jax.nn.sigmoid(gi[:, :HID] + gh[:, :HID])
        z = jax.nn.sigmoid(gi[:, HID:2 * HID] + gh[:, HID:2 * HID])
        n = jnp.tanh(gi[:, 2 * HID:] + r * gh[:, 2 * HID:])
        h_new = (1.0 - z) * n + z * h
        h_ref[...] = jnp.where(mask, h_new, h)

    step(xf_ref, wf16_ref, uf16_ref, bif_ref, bhf_ref, hf_ref)
    step(xr_ref, wb16_ref, ub16_ref, bib_ref, bhb_ref, hb_ref)

    @pl.when(t == QL - 1)
    def _():
        out_ref[:, :HID] = hf_ref[...]
        out_ref[:, HID:] = hb_ref[...]


def _gru_call(G, lens, Wf, Uf, bif, bhf, Wb, Ub, bib, bhb):
    def c2(shape):
        return pl.BlockSpec(shape, lambda t: (0, 0))

    return pl.pallas_call(
        _gru_body,
        grid=(QL,),
        in_specs=[
            pl.BlockSpec((SEQ, EPAD), lambda t: (t, 0)),
            pl.BlockSpec((SEQ, EPAD), lambda t: (t + QL, 0)),
            c2((SEQ, 1)),
            c2((EMB, 3 * HID)), c2((HID, 3 * HID)),
            c2((1, 3 * HID)), c2((1, 3 * HID)),
            c2((EMB, 3 * HID)), c2((HID, 3 * HID)),
            c2((1, 3 * HID)), c2((1, 3 * HID)),
        ],
        out_specs=pl.BlockSpec((SEQ, 2 * HID), lambda t: (0, 0)),
        out_shape=jax.ShapeDtypeStruct((SEQ, 2 * HID), jnp.float32),
        scratch_shapes=[pltpu.VMEM((SEQ, HID), jnp.float32),
                        pltpu.VMEM((SEQ, HID), jnp.float32),
                        pltpu.VMEM((EMB, 3 * HID), jnp.bfloat16),
                        pltpu.VMEM((HID, 3 * HID), jnp.bfloat16),
                        pltpu.VMEM((EMB, 3 * HID), jnp.bfloat16),
                        pltpu.VMEM((HID, 3 * HID), jnp.bfloat16)],
    )(G, G, lens, Wf, Uf, bif, bhf, Wb, Ub, bib, bhb)


def _adj_body(ge1_ref, ge2_ref, vgn_ref, qgn_ref,
              a1_ref, a2_ref, m12_ref, m21_ref):
    n1 = BB * KVG
    col1 = lax.broadcasted_iota(jnp.int32, (n1, n1), 1)
    acc1 = jnp.zeros((n1, n1), jnp.float32)
    for k in range(NBR):
        acc1 = acc1 + (ge1_ref[:, k:k + 1] == col1).astype(jnp.float32)
    a1_ref[...] = acc1 * (1.0 / NBR)

    n2 = BB * KQG
    col2 = lax.broadcasted_iota(jnp.int32, (n2, n2), 1)
    acc2 = jnp.zeros((n2, n2), jnp.float32)
    for k in range(NBR):
        acc2 = acc2 + (ge2_ref[:, k:k + 1] == col2).astype(jnp.float32)
    a2_ref[...] = acc2 * (1.0 / NBR)

    vs0 = jnp.sum(jnp.abs(vgn_ref[...]), axis=2) == 0.0
    qs0 = jnp.sum(jnp.abs(qgn_ref[...]), axis=2) == 0
    neg = jnp.float32(-1e9)
    rows12, rows21 = [], []
    for b in range(BB):
        mid12 = jnp.where(jnp.broadcast_to(qs0[b:b + 1, :], (KVG, KQG)), neg, 0.0)
        parts = []
        if b > 0:
            parts.append(jnp.full((KVG, KQG * b), neg, jnp.float32))
        parts.append(mid12)
        if b < BB - 1:
            parts.append(jnp.full((KVG, KQG * (BB - 1 - b)), neg, jnp.float32))
        rows12.append(jnp.concatenate(parts, axis=1) if len(parts) > 1 else parts[0])

        mid21 = jnp.where(jnp.broadcast_to(vs0[b:b + 1, :], (KQG, KVG)), neg, 0.0)
        parts = []
        if b > 0:
            parts.append(jnp.full((KQG, KVG * b), neg, jnp.float32))
        parts.append(mid21)
        if b < BB - 1:
            parts.append(jnp.full((KQG, KVG * (BB - 1 - b)), neg, jnp.float32))
        rows21.append(jnp.concatenate(parts, axis=1) if len(parts) > 1 else parts[0])
    m12_ref[...] = jnp.concatenate(rows12, axis=0)
    m21_ref[...] = jnp.concatenate(rows21, axis=0)


def _adj_call(ge1, ge2, vg_nodes, qg_nodes):
    return pl.pallas_call(
        _adj_body,
        out_shape=[
            jax.ShapeDtypeStruct((BB * KVG, BB * KVG), jnp.float32),
            jax.ShapeDtypeStruct((BB * KQG, BB * KQG), jnp.float32),
            jax.ShapeDtypeStruct((BB * KVG, BB * KQG), jnp.float32),
            jax.ShapeDtypeStruct((BB * KQG, BB * KVG), jnp.float32),
        ],
    )(ge1, ge2, vg_nodes, qg_nodes)


def _softmax_last(x):
    mx = jnp.max(x, axis=-1, keepdims=True)
    e = jnp.exp(x - mx)
    return e / jnp.sum(e, axis=-1, keepdims=True)


BNL = 256
NCT = DGM // BNL


def _layer_body(x1_ref, x2_ref, a1_ref, a2_ref, m12_ref, m21_ref,
                ws1_ref, wn1_ref, ws2_ref, wn2_ref, o1_ref, o2_ref,
                h1s_ref, h2s_ref, s12_ref, s21_ref):
    c = pl.program_id(0)
    dn = (((1,), (1,)), ((), ()))

    @pl.when(c < NCT)
    def _():
        x1 = x1_ref[...]
        x2 = x2_ref[...]
        z1 = jnp.dot(x1, wn1_ref[...], preferred_element_type=jnp.float32)
        h1c = jnp.maximum(
            jnp.dot(x1, ws1_ref[...], preferred_element_type=jnp.float32)
            + jnp.dot(a1_ref[...], z1, preferred_element_type=jnp.float32), 0.0)
        z2 = jnp.dot(x2, wn2_ref[...], preferred_element_type=jnp.float32)
        h2c = jnp.maximum(
            jnp.dot(x2, ws2_ref[...], preferred_element_type=jnp.float32)
            + jnp.dot(a2_ref[...], z2, preferred_element_type=jnp.float32), 0.0)
        h1s_ref[c] = h1c
        h2s_ref[c] = h2c
        ps12 = lax.dot_general(h1c, h2c, dn, preferred_element_type=jnp.float32)
        ps21 = lax.dot_general(h2c, h1c, dn, preferred_element_type=jnp.float32)

        @pl.when(c == 0)
        def _():
            s12_ref[...] = ps12
            s21_ref[...] = ps21

        @pl.when(c > 0)
        def _():
            s12_ref[...] += ps12
            s21_ref[...] += ps21

        @pl.when(c == NCT - 1)
        def _():
            scale = 1.0 / (DGM ** 0.5)
            s12_ref[...] = _softmax_last(s12_ref[...] * scale + m12_ref[...])
            s21_ref[...] = _softmax_last(s21_ref[...] * scale + m21_ref[...])

    @pl.when(c == NCT)
    def _():
        a12 = s12_ref[...]
        a21 = s21_ref[...]
        for cc in range(NCT):
            h1c = h1s_ref[cc]
            h2c = h2s_ref[cc]
            o1_ref[:, cc * BNL:(cc + 1) * BNL] = h1c + jnp.dot(
                a12, h2c, preferred_element_type=jnp.float32)
            o2_ref[:, cc * BNL:(cc + 1) * BNL] = h2c + jnp.dot(
                a21, h1c, preferred_element_type=jnp.float32)


def _layer(x1, x2, a1, a2, m12, m21, ws1, wn1, ws2, wn2):
    m1, d = x1.shape
    m2 = x2.shape[0]

    def const(shape):
        return pl.BlockSpec(shape, lambda c: (0, 0))

    def wspec(shape):
        return pl.BlockSpec(shape, lambda c: (0, jnp.minimum(c, NCT - 1)))

    return pl.pallas_call(
        _layer_body,
        grid=(NCT + 1,),
        in_specs=[
            const((m1, d)), const((m2, d)),
            const((m1, m1)), const((m2, m2)),
            const((m1, m2)), const((m2, m1)),
            wspec((d, BNL)), wspec((d, BNL)),
            wspec((d, BNL)), wspec((d, BNL)),
        ],
        out_specs=[
            pl.BlockSpec((m1, DGM), lambda c: (0, 0)),
            pl.BlockSpec((m2, DGM), lambda c: (0, 0)),
        ],
        out_shape=[
            jax.ShapeDtypeStruct((m1, DGM), jnp.float32),
            jax.ShapeDtypeStruct((m2, DGM), jnp.float32),
        ],
        scratch_shapes=[
            pltpu.VMEM((NCT, m1, BNL), jnp.float32),
            pltpu.VMEM((NCT, m2, BNL), jnp.float32),
            pltpu.VMEM((m1, m2), jnp.float32),
            pltpu.VMEM((m2, m1), jnp.float32),
        ],
    )(x1, x2, a1, a2, m12, m21, ws1, wn1, ws2, wn2)


def _head1_body(x2_ref, q_ref, w1_ref, b1_ref, o_ref):
    ffs = [jnp.max(x2_ref[b * KQG:(b + 1) * KQG, :], axis=0, keepdims=True)
           for b in range(BB)]
    ff = jnp.concatenate(ffs, axis=0)
    h = jnp.maximum(q_ref[...], 0.0) * ff
    o_ref[...] = jnp.maximum(
        jnp.dot(h, w1_ref[...], preferred_element_type=jnp.float32) + b1_ref[...],
        0.0)


def _head1(x2, qenc, w1, b1):
    bn = 512
    g = (OUTD + bn - 1) // bn
    return pl.pallas_call(
        _head1_body,
        grid=(g,),
        in_specs=[
            pl.BlockSpec((BB * KQG, DGM), lambda c: (0, 0)),
            pl.BlockSpec((BB, DGM), lambda c: (0, 0)),
            pl.BlockSpec((DGM, bn), lambda c: (0, c)),
            pl.BlockSpec((1, bn), lambda c: (0, c)),
        ],
        out_specs=pl.BlockSpec((BB, bn), lambda c: (0, c)),
        out_shape=jax.ShapeDtypeStruct((BB, OUTD), jnp.float32),
    )(x2, qenc, w1, b1)


def _head2_body(h_ref, w2_ref, b2_ref, o_ref):
    o_ref[...] = jnp.dot(h_ref[...], w2_ref[...],
                         preferred_element_type=jnp.float32) + b2_ref[...]


def _head2(hid1, w2, b2):
    bn = 512
    g = (OUTD + bn - 1) // bn
    return pl.pallas_call(
        _head2_body,
        grid=(g,),
        in_specs=[
            pl.BlockSpec((BB, OUTD), lambda c: (0, 0)),
            pl.BlockSpec((OUTD, bn), lambda c: (0, c)),
            pl.BlockSpec((1, bn), lambda c: (0, c)),
        ],
        out_specs=pl.BlockSpec((BB, bn), lambda c: (0, c)),
        out_shape=jax.ShapeDtypeStruct((BB, OUTD), jnp.float32),
    )(hid1, w2, b2)


def kernel(question, vg_nodes, vg_edges, qg_nodes, qg_edges, qglen, qlen,
           emb_table, Wf, Uf, bif, bhf, Wb, Ub, bib, bhb,
           Ws1a, Wn1a, Ws2a, Wn2a, Ws1b, Wn1b, Ws2b, Wn2b, W1, b1, W2, b2):
    tokq = question.astype(jnp.int32)
    tokg = qg_nodes.reshape(BB * KQG, NWORD).astype(jnp.int32)
    tokg = jnp.pad(tokg, ((0, 0), (0, QL - NWORD)))
    tok_f = jnp.concatenate([tokq, tokg], axis=0)
    lens = jnp.concatenate([qlen.astype(jnp.int32),
                            qglen.reshape(-1).astype(jnp.int32)])
    tt = jnp.arange(QL, dtype=jnp.int32)
    pos = jnp.clip(lens[:, None] - 1 - tt[None, :], 0, QL - 1)
    tok_r = jnp.take_along_axis(tok_f, pos, axis=1)
    ids = jnp.concatenate([
        tok_f.T.reshape(-1), tok_r.T.reshape(-1),
        jnp.zeros((NIDS_PAD - NIDS,), jnp.int32)]).astype(jnp.int32)
    tpad = _pad_table(emb_table)
    G = _sc_gather(tpad, ids)

    H = _gru_call(G, lens.reshape(SEQ, 1),
                  Wf, Uf, bif.reshape(1, -1), bhf.reshape(1, -1),
                  Wb, Ub, bib.reshape(1, -1), bhb.reshape(1, -1))
    qenc = H[:BB]
    qg_enc = H[BB:]

    roff1 = (jnp.arange(BB * KVG, dtype=jnp.int32) // KVG * KVG)[:, None]
    ge1 = vg_edges.reshape(BB * KVG, NBR).astype(jnp.int32) + roff1
    roff2 = (jnp.arange(BB * KQG, dtype=jnp.int32) // KQG * KQG)[:, None]
    ge2 = qg_edges.reshape(BB * KQG, NBR).astype(jnp.int32) + roff2
    A1, A2, M12, M21 = _adj_call(ge1, ge2, vg_nodes, qg_nodes.astype(jnp.int32))

    qb1 = jnp.broadcast_to(qenc[:, None, :], (BB, KVG, DGM)).reshape(BB * KVG, DGM)
    qb2 = jnp.broadcast_to(qenc[:, None, :], (BB, KQG, DGM)).reshape(BB * KQG, DGM)
    x1 = jnp.concatenate([vg_nodes.reshape(BB * KVG, DVG), qb1], axis=1)
    x2 = jnp.concatenate([qg_enc, qb2], axis=1)

    x1, x2 = _layer(x1, x2, A1, A2, M12, M21, Ws1a, Wn1a, Ws2a, Wn2a)
    x1, x2 = _layer(x1, x2, A1, A2, M12, M21, Ws1b, Wn1b, Ws2b, Wn2b)

    hid1 = _head1(x2, qenc, W1, b1.reshape(1, OUTD))
    return _head2(hid1, W2, b2.reshape(1, OUTD))


# speedup vs baseline: 1.0258x; 1.0009x over previous
import functools

import jax
import jax.numpy as jnp
from jax import lax
from jax.experimental import pallas as pl
from jax.experimental.pallas import tpu as pltpu
from jax.experimental.pallas import tpu_sc as plsc

BB = 8
QL = 14
KVG = 36
KQG = 14
NWORD = 10
NBR = 4
EMB = 300
HID = 1024
DVG = 2048
DGM = 2048
OUTD = 3129
SEQ = BB + BB * KQG
EPAD = 384
NIDS = 2 * SEQ * QL
NIDS_PAD = 3584
PER_TILE = NIDS_PAD // 32


def _sc_gather(table_pad, ids):
    mesh = plsc.VectorSubcoreMesh(core_axis_name="c", subcore_axis_name="s")

    @functools.partial(
        pl.kernel,
        mesh=mesh,
        out_type=jax.ShapeDtypeStruct((NIDS_PAD, EPAD), jnp.float32),
        scratch_types=[
            pltpu.VMEM((PER_TILE,), jnp.int32),
            pltpu.VMEM((PER_TILE, EPAD), jnp.float32),
            pltpu.SemaphoreType.DMA,
        ],
    )
    def gk(table_hbm, idx_hbm, out_hbm, idx_v, rows_v, sem):
        wid = lax.axis_index("s") * 2 + lax.axis_index("c")
        base = wid * PER_TILE
        pltpu.sync_copy(idx_hbm.at[pl.ds(base, PER_TILE)], idx_v)
        ch = 16
        cps = [pltpu.async_copy(table_hbm.at[idx_v.at[pl.ds(k * ch, ch)]],
                                rows_v.at[pl.ds(k * ch, ch)], sem)
               for k in range(PER_TILE // ch)]
        for cp in cps:
            cp.wait()
        pltpu.sync_copy(rows_v, out_hbm.at[pl.ds(base, PER_TILE)])

    return gk(table_pad, ids)


def _pad_body(x_ref, o_ref):
    o_ref[...] = jnp.concatenate(
        [x_ref[...], jnp.zeros((x_ref.shape[0], EPAD - EMB), jnp.float32)],
        axis=1)


def _pad_table(table):
    rows = table.shape[0]
    rt = 2000
    return pl.pallas_call(
        _pad_body,
        grid=(rows // rt,),
        in_specs=[pl.BlockSpec((rt, EMB), lambda r: (r, 0))],
        out_specs=pl.BlockSpec((rt, EPAD), lambda r: (r, 0)),
        out_shape=jax.ShapeDtypeStruct((rows, EPAD), jnp.float32),
    )(table)


def _gru_body(xf_ref, xr_ref, lens_ref, wf_ref, uf_ref, bif_ref, bhf_ref,
              wb_ref, ub_ref, bib_ref, bhb_ref, out_ref, hf_ref, hb_ref,
              wf16_ref, uf16_ref, wb16_ref, ub16_ref):
    t = pl.program_id(0)

    @pl.when(t == 0)
    def _():
        hf_ref[...] = jnp.zeros_like(hf_ref)
        hb_ref[...] = jnp.zeros_like(hb_ref)
        wf16_ref[...] = wf_ref[...].astype(jnp.bfloat16)
        uf16_ref[...] = uf_ref[...].astype(jnp.bfloat16)
        wb16_ref[...] = wb_ref[...].astype(jnp.bfloat16)
        ub16_ref[...] = ub_ref[...].astype(jnp.bfloat16)

    mask = lens_ref[...] > t

    def step(x_ref, w_ref, u_ref, bi_ref, bh_ref, h_ref):
        x = x_ref[:, :EMB].astype(jnp.bfloat16)
        h = h_ref[...]
        h16 = h.astype(jnp.bfloat16)
        gi = jnp.dot(x, w_ref[...], preferred_element_type=jnp.float32) + bi_ref[...]
        gh = jnp.dot(h16, u_ref[...], preferred_element_type=jnp.float32) + bh_ref[...]
        r = jax.nn.sigmoid(gi[:, :HID] + gh[:, :HID])
        z = jax.nn.sigmoid(gi[:, HID:2 * HID] + gh[:, HID:2 * HID])
        n = jnp.tanh(gi[:, 2 * HID:] + r * gh[:, 2 * HID:])
        h_new = (1.0 - z) * n + z * h
        h_ref[...] = jnp.where(mask, h_new, h)

    step(xf_ref, wf16_ref, uf16_ref, bif_ref, bhf_ref, hf_ref)
    step(xr_ref, wb16_ref, ub16_ref, bib_ref, bhb_ref, hb_ref)

    @pl.when(t == QL - 1)
    def _():
        out_ref[:, :HID] = hf_ref[...]
        out_ref[:, HID:] = hb_ref[...]


def _gru_call(G, lens, Wf, Uf, bif, bhf, Wb, Ub, bib, bhb):
    def c2(shape):
        return pl.BlockSpec(shape, lambda t: (0, 0))

    return pl.pallas_call(
        _gru_body,
        grid=(QL,),
        in_specs=[
            pl.BlockSpec((SEQ, EPAD), lambda t: (t, 0)),
            pl.BlockSpec((SEQ, EPAD), lambda t: (t + QL, 0)),
            c2((SEQ, 1)),
            c2((EMB, 3 * HID)), c2((HID, 3 * HID)),
            c2((1, 3 * HID)), c2((1, 3 * HID)),
            c2((EMB, 3 * HID)), c2((HID, 3 * HID)),
            c2((1, 3 * HID)), c2((1, 3 * HID)),
        ],
        out_specs=pl.BlockSpec((SEQ, 2 * HID), lambda t: (0, 0)),
        out_shape=jax.ShapeDtypeStruct((SEQ, 2 * HID), jnp.float32),
        scratch_shapes=[pltpu.VMEM((SEQ, HID), jnp.float32),
                        pltpu.VMEM((SEQ, HID), jnp.float32),
                        pltpu.VMEM((EMB, 3 * HID), jnp.bfloat16),
                        pltpu.VMEM((HID, 3 * HID), jnp.bfloat16),
                        pltpu.VMEM((EMB, 3 * HID), jnp.bfloat16),
                        pltpu.VMEM((HID, 3 * HID), jnp.bfloat16)],
    )(G, G, lens, Wf, Uf, bif, bhf, Wb, Ub, bib, bhb)


def _adj_body(ge1_ref, ge2_ref, vgn_ref, qgn_ref,
              a1_ref, a2_ref, m12_ref, m21_ref):
    n1 = BB * KVG
    col1 = lax.broadcasted_iota(jnp.int32, (n1, n1), 1)
    acc1 = jnp.zeros((n1, n1), jnp.float32)
    for k in range(NBR):
        acc1 = acc1 + (ge1_ref[:, k:k + 1] == col1).astype(jnp.float32)
    a1_ref[...] = acc1 * (1.0 / NBR)

    n2 = BB * KQG
    col2 = lax.broadcasted_iota(jnp.int32, (n2, n2), 1)
    acc2 = jnp.zeros((n2, n2), jnp.float32)
    for k in range(NBR):
        acc2 = acc2 + (ge2_ref[:, k:k + 1] == col2).astype(jnp.float32)
    a2_ref[...] = acc2 * (1.0 / NBR)

    vs0 = jnp.sum(jnp.abs(vgn_ref[...]), axis=2) == 0.0
    qs0 = jnp.sum(jnp.abs(qgn_ref[...]), axis=2) == 0
    neg = jnp.float32(-1e9)
    rows12, rows21 = [], []
    for b in range(BB):
        mid12 = jnp.where(jnp.broadcast_to(qs0[b:b + 1, :], (KVG, KQG)), neg, 0.0)
        parts = []
        if b > 0:
            parts.append(jnp.full((KVG, KQG * b), neg, jnp.float32))
        parts.append(mid12)
        if b < BB - 1:
            parts.append(jnp.full((KVG, KQG * (BB - 1 - b)), neg, jnp.float32))
        rows12.append(jnp.concatenate(parts, axis=1) if len(parts) > 1 else parts[0])

        mid21 = jnp.where(jnp.broadcast_to(vs0[b:b + 1, :], (KQG, KVG)), neg, 0.0)
        parts = []
        if b > 0:
            parts.append(jnp.full((KQG, KVG * b), neg, jnp.float32))
        parts.append(mid21)
        if b < BB - 1:
            parts.append(jnp.full((KQG, KVG * (BB - 1 - b)), neg, jnp.float32))
        rows21.append(jnp.concatenate(parts, axis=1) if len(parts) > 1 else parts[0])
    m12_ref[...] = jnp.concatenate(rows12, axis=0)
    m21_ref[...] = jnp.concatenate(rows21, axis=0)


def _adj_call(ge1, ge2, vg_nodes, qg_nodes):
    return pl.pallas_call(
        _adj_body,
        out_shape=[
            jax.ShapeDtypeStruct((BB * KVG, BB * KVG), jnp.float32),
            jax.ShapeDtypeStruct((BB * KQG, BB * KQG), jnp.float32),
            jax.ShapeDtypeStruct((BB * KVG, BB * KQG), jnp.float32),
            jax.ShapeDtypeStruct((BB * KQG, BB * KVG), jnp.float32),
        ],
    )(ge1, ge2, vg_nodes, qg_nodes)


def _softmax_last(x):
    mx = jnp.max(x, axis=-1, keepdims=True)
    e = jnp.exp(x - mx)
    return e / jnp.sum(e, axis=-1, keepdims=True)


BNL = 256
NCT = DGM // BNL


def _layer_body(x1_ref, x2_ref, a1_ref, a2_ref, m12_ref, m21_ref,
                ws1_ref, wn1_ref, ws2_ref, wn2_ref, o1_ref, o2_ref,
                h1s_ref, h2s_ref, s12_ref, s21_ref):
    c = pl.program_id(0)
    dn = (((1,), (1,)), ((), ()))

    @pl.when(c < NCT)
    def _():
        x1 = x1_ref[...]
        x2 = x2_ref[...]
        z1 = jnp.dot(x1, wn1_ref[...], preferred_element_type=jnp.float32)
        h1c = jnp.maximum(
            jnp.dot(x1, ws1_ref[...], preferred_element_type=jnp.float32)
            + jnp.dot(a1_ref[...], z1, preferred_element_type=jnp.float32), 0.0)
        z2 = jnp.dot(x2, wn2_ref[...], preferred_element_type=jnp.float32)
        h2c = jnp.maximum(
            jnp.dot(x2, ws2_ref[...], preferred_element_type=jnp.float32)
            + jnp.dot(a2_ref[...], z2, preferred_element_type=jnp.float32), 0.0)
        h1s_ref[c] = h1c
        h2s_ref[c] = h2c
        ps12 = lax.dot_general(h1c, h2c, dn, preferred_element_type=jnp.float32)
        ps21 = lax.dot_general(h2c, h1c, dn, preferred_element_type=jnp.float32)

        @pl.when(c == 0)
        def _():
            s12_ref[...] = ps12
            s21_ref[...] = ps21

        @pl.when(c > 0)
        def _():
            s12_ref[...] += ps12
            s21_ref[...] += ps21

        @pl.when(c == NCT - 1)
        def _():
            scale = 1.0 / (DGM ** 0.5)
            s12_ref[...] = _softmax_last(s12_ref[...] * scale + m12_ref[...])
            s21_ref[...] = _softmax_last(s21_ref[...] * scale + m21_ref[...])

    @pl.when(c == NCT)
    def _():
        a12 = s12_ref[...]
        a21 = s21_ref[...]
        for cc in range(NCT):
            h1c = h1s_ref[cc]
            h2c = h2s_ref[cc]
            o1_ref[:, cc * BNL:(cc + 1) * BNL] = h1c + jnp.dot(
                a12, h2c, preferred_element_type=jnp.float32)
            o2_ref[:, cc * BNL:(cc + 1) * BNL] = h2c + jnp.dot(
                a21, h1c, preferred_element_type=jnp.float32)


def _layer(x1, x2, a1, a2, m12, m21, ws1, wn1, ws2, wn2):
    m1, d = x1.shape
    m2 = x2.shape[0]

    def const(shape):
        return pl.BlockSpec(shape, lambda c: (0, 0))

    def wspec(shape):
        return pl.BlockSpec(shape, lambda c: (0, jnp.minimum(c, NCT - 1)))

    return pl.pallas_call(
        _layer_body,
        grid=(NCT + 1,),
        in_specs=[
            const((m1, d)), const((m2, d)),
            const((m1, m1)), const((m2, m2)),
            const((m1, m2)), const((m2, m1)),
            wspec((d, BNL)), wspec((d, BNL)),
            wspec((d, BNL)), wspec((d, BNL)),
        ],
        out_specs=[
            pl.BlockSpec((m1, DGM), lambda c: (0, 0)),
            pl.BlockSpec((m2, DGM), lambda c: (0, 0)),
        ],
        out_shape=[
            jax.ShapeDtypeStruct((m1, DGM), jnp.float32),
            jax.ShapeDtypeStruct((m2, DGM), jnp.float32),
        ],
        scratch_shapes=[
            pltpu.VMEM((NCT, m1, BNL), jnp.float32),
            pltpu.VMEM((NCT, m2, BNL), jnp.float32),
            pltpu.VMEM((m1, m2), jnp.float32),
            pltpu.VMEM((m2, m1), jnp.float32),
        ],
    )(x1, x2, a1, a2, m12, m21, ws1, wn1, ws2, wn2)


def _head1_body(x2_ref, q_ref, w1_ref, b1_ref, o_ref):
    ffs = [jnp.max(x2_ref[b * KQG:(b + 1) * KQG, :], axis=0, keepdims=True)
           for b in range(BB)]
    ff = jnp.concatenate(ffs, axis=0)
    h = jnp.maximum(q_ref[...], 0.0) * ff
    o_ref[...] = jnp.maximum(
        jnp.dot(h, w1_ref[...], preferred_element_type=jnp.float32) + b1_ref[...],
        0.0)


def _head1(x2, qenc, w1, b1):
    bn = 512
    g = (OUTD + bn - 1) // bn
    return pl.pallas_call(
        _head1_body,
        grid=(g,),
        in_specs=[
            pl.BlockSpec((BB * KQG, DGM), lambda c: (0, 0)),
            pl.BlockSpec((BB, DGM), lambda c: (0, 0)),
            pl.BlockSpec((DGM, bn), lambda c: (0, c)),
            pl.BlockSpec((1, bn), lambda c: (0, c)),
        ],
        out_specs=pl.BlockSpec((BB, bn), lambda c: (0, c)),
        out_shape=jax.ShapeDtypeStruct((BB, OUTD), jnp.float32),
    )(x2, qenc, w1, b1)


def _head2_body(h_ref, w2_ref, b2_ref, o_ref):
    o_ref[...] = jnp.dot(h_ref[...], w2_ref[...],
                         preferred_element_type=jnp.float32) + b2_ref[...]


def _head2(hid1, w2, b2):
    bn = 512
    g = (OUTD + bn - 1) // bn
    return pl.pallas_call(
        _head2_body,
        grid=(g,),
        in_specs=[
            pl.BlockSpec((BB, OUTD), lambda c: (0, 0)),
            pl.BlockSpec((OUTD, bn), lambda c: (0, c)),
            pl.BlockSpec((1, bn), lambda c: (0, c)),
        ],
        out_specs=pl.BlockSpec((BB, bn), lambda c: (0, c)),
        out_shape=jax.ShapeDtypeStruct((BB, OUTD), jnp.float32),
    )(hid1, w2, b2)


def kernel(question, vg_nodes, vg_edges, qg_nodes, qg_edges, qglen, qlen,
           emb_table, Wf, Uf, bif, bhf, Wb, Ub, bib, bhb,
           Ws1a, Wn1a, Ws2a, Wn2a, Ws1b, Wn1b, Ws2b, Wn2b, W1, b1, W2, b2):
    tokq = question.astype(jnp.int32)
    tokg = qg_nodes.reshape(BB * KQG, NWORD).astype(jnp.int32)
    tokg = jnp.pad(tokg, ((0, 0), (0, QL - NWORD)))
    tok_f = jnp.concatenate([tokq, tokg], axis=0)
    lens = jnp.concatenate([qlen.astype(jnp.int32),
                            qglen.reshape(-1).astype(jnp.int32)])
    tt = jnp.arange(QL, dtype=jnp.int32)
    pos = jnp.clip(lens[:, None] - 1 - tt[None, :], 0, QL - 1)
    tok_r = jnp.take_along_axis(tok_f, pos, axis=1)
    ids = jnp.concatenate([
        tok_f.T.reshape(-1), tok_r.T.reshape(-1),
        jnp.zeros((NIDS_PAD - NIDS,), jnp.int32)]).astype(jnp.int32)
    tpad = _pad_table(emb_table)
    G = _sc_gather(tpad, ids)

    H = _gru_call(G, lens.reshape(SEQ, 1),
                  Wf, Uf, bif.reshape(1, -1), bhf.reshape(1, -1),
                  Wb, Ub, bib.reshape(1, -1), bhb.reshape(1, -1))
    qenc = H[:BB]
    qg_enc = H[BB:]

    roff1 = (jnp.arange(BB * KVG, dtype=jnp.int32) // KVG * KVG)[:, None]
    ge1 = vg_edges.reshape(BB * KVG, NBR).astype(jnp.int32) + roff1
    roff2 = (jnp.arange(BB * KQG, dtype=jnp.int32) // KQG * KQG)[:, None]
    ge2 = qg_edges.reshape(BB * KQG, NBR).astype(jnp.int32) + roff2
    A1, A2, M12, M21 = _adj_call(ge1, ge2, vg_nodes, qg_nodes.astype(jnp.int32))

    qb1 = jnp.broadcast_to(qenc[:, None, :], (BB, KVG, DGM)).reshape(BB * KVG, DGM)
    qb2 = jnp.broadcast_to(qenc[:, None, :], (BB, KQG, DGM)).reshape(BB * KQG, DGM)
    x1 = jnp.concatenate([vg_nodes.reshape(BB * KVG, DVG), qb1], axis=1)
    x2 = jnp.concatenate([qg_enc, qb2], axis=1)

    x1, x2 = _layer(x1, x2, A1, A2, M12, M21, Ws1a, Wn1a, Ws2a, Wn2a)
    x1, x2 = _layer(x1, x2, A1, A2, M12, M21, Ws1b, Wn1b, Ws2b, Wn2b)

    hid1 = _head1(x2, qenc, W1, b1.reshape(1, OUTD))
    return _head2(hid1, W2, b2.reshape(1, OUTD))


# speedup vs baseline: 1.0450x; 1.0187x over previous
import functools

import jax
import jax.numpy as jnp
from jax import lax
from jax.experimental import pallas as pl
from jax.experimental.pallas import tpu as pltpu
from jax.experimental.pallas import tpu_sc as plsc

BB = 8
QL = 14
KVG = 36
KQG = 14
NWORD = 10
NBR = 4
EMB = 300
HID = 1024
DVG = 2048
DGM = 2048
OUTD = 3129
SEQ = BB + BB * KQG
EPAD = 384
NIDS = 2 * SEQ * QL
NIDS_PAD = 3584
PER_TILE = NIDS_PAD // 32


def _sc_gather(table_pad, ids):
    mesh = plsc.VectorSubcoreMesh(core_axis_name="c", subcore_axis_name="s")

    @functools.partial(
        pl.kernel,
        mesh=mesh,
        out_type=jax.ShapeDtypeStruct((NIDS_PAD, EPAD), jnp.float32),
        scratch_types=[
            pltpu.VMEM((PER_TILE,), jnp.int32),
            pltpu.VMEM((PER_TILE, EPAD), jnp.float32),
            pltpu.SemaphoreType.DMA,
        ],
    )
    def gk(table_hbm, idx_hbm, out_hbm, idx_v, rows_v, sem):
        wid = lax.axis_index("s") * 2 + lax.axis_index("c")
        base = wid * PER_TILE
        pltpu.sync_copy(idx_hbm.at[pl.ds(base, PER_TILE)], idx_v)
        ch = 16
        cps = [pltpu.async_copy(table_hbm.at[idx_v.at[pl.ds(k * ch, ch)]],
                                rows_v.at[pl.ds(k * ch, ch)], sem)
               for k in range(PER_TILE // ch)]
        for cp in cps:
            cp.wait()
        pltpu.sync_copy(rows_v, out_hbm.at[pl.ds(base, PER_TILE)])

    return gk(table_pad, ids)


def _pad_body(x_ref, o_ref):
    o_ref[...] = jnp.concatenate(
        [x_ref[...], jnp.zeros((x_ref.shape[0], EPAD - EMB), jnp.float32)],
        axis=1)


def _pad_table(table):
    rows = table.shape[0]
    rt = 2000
    return pl.pallas_call(
        _pad_body,
        grid=(rows // rt,),
        in_specs=[pl.BlockSpec((rt, EMB), lambda r: (r, 0))],
        out_specs=pl.BlockSpec((rt, EPAD), lambda r: (r, 0)),
        out_shape=jax.ShapeDtypeStruct((rows, EPAD), jnp.float32),
    )(table)


def _gru_body(xf_ref, xr_ref, lens_ref, wf_ref, uf_ref, bif_ref, bhf_ref,
              wb_ref, ub_ref, bib_ref, bhb_ref, out_ref, hf_ref, hb_ref,
              wf16_ref, uf16_ref, wb16_ref, ub16_ref):
    t = pl.program_id(0)

    @pl.when(t == 0)
    def _():
        hf_ref[...] = jnp.zeros_like(hf_ref)
        hb_ref[...] = jnp.zeros_like(hb_ref)
        wf16_ref[...] = wf_ref[...].astype(jnp.bfloat16)
        uf16_ref[...] = uf_ref[...].astype(jnp.bfloat16)
        wb16_ref[...] = wb_ref[...].astype(jnp.bfloat16)
        ub16_ref[...] = ub_ref[...].astype(jnp.bfloat16)

    mask = lens_ref[...] > t

    def step(x_ref, w_ref, u_ref, bi_ref, bh_ref, h_ref):
        x = x_ref[:, :EMB].astype(jnp.bfloat16)
        h = h_ref[...]
        h16 = h.astype(jnp.bfloat16)
        gi = jnp.dot(x, w_ref[...], preferred_element_type=jnp.float32) + bi_ref[...]
        gh = jnp.dot(h16, u_ref[...], preferred_element_type=jnp.float32) + bh_ref[...]
        r = jax.nn.sigmoid(gi[:, :HID] + gh[:, :HID])
        z = jax.nn.sigmoid(gi[:, HID:2 * HID] + gh[:, HID:2 * HID])
        n = jnp.tanh(gi[:, 2 * HID:] + r * gh[:, 2 * HID:])
        h_new = (1.0 - z) * n + z * h
        h_ref[...] = jnp.where(mask, h_new, h)

    step(xf_ref, wf16_ref, uf16_ref, bif_ref, bhf_ref, hf_ref)
    step(xr_ref, wb16_ref, ub16_ref, bib_ref, bhb_ref, hb_ref)

    @pl.when(t == QL - 1)
    def _():
        out_ref[:, :HID] = hf_ref[...]
        out_ref[:, HID:] = hb_ref[...]


def _gru_call(G, lens, Wf, Uf, bif, bhf, Wb, Ub, bib, bhb):
    def c2(shape):
        return pl.BlockSpec(shape, lambda t: (0, 0))

    return pl.pallas_call(
        _gru_body,
        grid=(QL,),
        in_specs=[
            pl.BlockSpec((SEQ, EPAD), lambda t: (t, 0)),
            pl.BlockSpec((SEQ, EPAD), lambda t: (t + QL, 0)),
            c2((SEQ, 1)),
            c2((EMB, 3 * HID)), c2((HID, 3 * HID)),
            c2((1, 3 * HID)), c2((1, 3 * HID)),
            c2((EMB, 3 * HID)), c2((HID, 3 * HID)),
            c2((1, 3 * HID)), c2((1, 3 * HID)),
        ],
        out_specs=pl.BlockSpec((SEQ, 2 * HID), lambda t: (0, 0)),
        out_shape=jax.ShapeDtypeStruct((SEQ, 2 * HID), jnp.float32),
        scratch_shapes=[pltpu.VMEM((SEQ, HID), jnp.float32),
                        pltpu.VMEM((SEQ, HID), jnp.float32),
                        pltpu.VMEM((EMB, 3 * HID), jnp.bfloat16),
                        pltpu.VMEM((HID, 3 * HID), jnp.bfloat16),
                        pltpu.VMEM((EMB, 3 * HID), jnp.bfloat16),
                        pltpu.VMEM((HID, 3 * HID), jnp.bfloat16)],
    )(G, G, lens, Wf, Uf, bif, bhf, Wb, Ub, bib, bhb)


def _adj_body(ge1_ref, ge2_ref, vgn_ref, qgn_ref,
              a1_ref, a2_ref, m12_ref, m21_ref):
    n1 = BB * KVG
    col1 = lax.broadcasted_iota(jnp.int32, (n1, n1), 1)
    acc1 = jnp.zeros((n1, n1), jnp.float32)
    for k in range(NBR):
        acc1 = acc1 + (ge1_ref[:, k:k + 1] == col1).astype(jnp.float32)
    a1_ref[...] = acc1 * (1.0 / NBR)

    n2 = BB * KQG
    col2 = lax.broadcasted_iota(jnp.int32, (n2, n2), 1)
    acc2 = jnp.zeros((n2, n2), jnp.float32)
    for k in range(NBR):
        acc2 = acc2 + (ge2_ref[:, k:k + 1] == col2).astype(jnp.float32)
    a2_ref[...] = acc2 * (1.0 / NBR)

    vs0 = jnp.sum(jnp.abs(vgn_ref[...]), axis=2) == 0.0
    qs0 = jnp.sum(jnp.abs(qgn_ref[...]), axis=2) == 0
    neg = jnp.float32(-1e9)
    rows12, rows21 = [], []
    for b in range(BB):
        mid12 = jnp.where(jnp.broadcast_to(qs0[b:b + 1, :], (KVG, KQG)), neg, 0.0)
        parts = []
        if b > 0:
            parts.append(jnp.full((KVG, KQG * b), neg, jnp.float32))
        parts.append(mid12)
        if b < BB - 1:
            parts.append(jnp.full((KVG, KQG * (BB - 1 - b)), neg, jnp.float32))
        rows12.append(jnp.concatenate(parts, axis=1) if len(parts) > 1 else parts[0])

        mid21 = jnp.where(jnp.broadcast_to(vs0[b:b + 1, :], (KQG, KVG)), neg, 0.0)
        parts = []
        if b > 0:
            parts.append(jnp.full((KQG, KVG * b), neg, jnp.float32))
        parts.append(mid21)
        if b < BB - 1:
            parts.append(jnp.full((KQG, KVG * (BB - 1 - b)), neg, jnp.float32))
        rows21.append(jnp.concatenate(parts, axis=1) if len(parts) > 1 else parts[0])
    m12_ref[...] = jnp.concatenate(rows12, axis=0)
    m21_ref[...] = jnp.concatenate(rows21, axis=0)


def _adj_call(ge1, ge2, vg_nodes, qg_nodes):
    return pl.pallas_call(
        _adj_body,
        out_shape=[
            jax.ShapeDtypeStruct((BB * KVG, BB * KVG), jnp.float32),
            jax.ShapeDtypeStruct((BB * KQG, BB * KQG), jnp.float32),
            jax.ShapeDtypeStruct((BB * KVG, BB * KQG), jnp.float32),
            jax.ShapeDtypeStruct((BB * KQG, BB * KVG), jnp.float32),
        ],
    )(ge1, ge2, vg_nodes, qg_nodes)


def _softmax_last(x):
    mx = jnp.max(x, axis=-1, keepdims=True)
    e = jnp.exp(x - mx)
    return e / jnp.sum(e, axis=-1, keepdims=True)


BNL = 256
NCT = DGM // BNL


def _layer_body(x1_ref, x2_ref, a1_ref, a2_ref, m12_ref, m21_ref,
                ws1_ref, wn1_ref, ws2_ref, wn2_ref, o1_ref, o2_ref,
                h1s_ref, h2s_ref, s12_ref, s21_ref):
    c = pl.program_id(0)
    dn = (((1,), (1,)), ((), ()))

    @pl.when(c < NCT)
    def _():
        x1 = x1_ref[...]
        x2 = x2_ref[...]
        z1 = jnp.dot(x1, wn1_ref[...], preferred_element_type=jnp.float32)
        h1c = jnp.maximum(
            jnp.dot(x1, ws1_ref[...], preferred_element_type=jnp.float32)
            + jnp.dot(a1_ref[...], z1, preferred_element_type=jnp.float32), 0.0)
        z2 = jnp.dot(x2, wn2_ref[...], preferred_element_type=jnp.float32)
        h2c = jnp.maximum(
            jnp.dot(x2, ws2_ref[...], preferred_element_type=jnp.float32)
            + jnp.dot(a2_ref[...], z2, preferred_element_type=jnp.float32), 0.0)
        h1s_ref[c] = h1c
        h2s_ref[c] = h2c
        ps12 = lax.dot_general(h1c, h2c, dn, preferred_element_type=jnp.float32)
        ps21 = lax.dot_general(h2c, h1c, dn, preferred_element_type=jnp.float32)

        @pl.when(c == 0)
        def _():
            s12_ref[...] = ps12
            s21_ref[...] = ps21

        @pl.when(c > 0)
        def _():
            s12_ref[...] += ps12
            s21_ref[...] += ps21

        @pl.when(c == NCT - 1)
        def _():
            scale = 1.0 / (DGM ** 0.5)
            s12_ref[...] = _softmax_last(s12_ref[...] * scale + m12_ref[...])
            s21_ref[...] = _softmax_last(s21_ref[...] * scale + m21_ref[...])

    @pl.when(c == NCT)
    def _():
        a12 = s12_ref[...]
        a21 = s21_ref[...]
        for cc in range(NCT):
            h1c = h1s_ref[cc]
            h2c = h2s_ref[cc]
            o1_ref[:, cc * BNL:(cc + 1) * BNL] = h1c + jnp.dot(
                a12, h2c, preferred_element_type=jnp.float32)
            o2_ref[:, cc * BNL:(cc + 1) * BNL] = h2c + jnp.dot(
                a21, h1c, preferred_element_type=jnp.float32)


def _layer_a_body(vg_ref, qg_ref, q_ref, a1_ref, a2_ref, m12_ref, m21_ref,
                  ws1t_ref, ws1b_ref, wn1t_ref, wn1b_ref,
                  ws2t_ref, ws2b_ref, wn2t_ref, wn2b_ref,
                  o1_ref, o2_ref, h1s_ref, h2s_ref, s12_ref, s21_ref):
    c = pl.program_id(0)
    dn = (((1,), (1,)), ((), ()))
    m1 = BB * KVG
    m2 = BB * KQG

    @pl.when(c < NCT)
    def _():
        e1 = (lax.broadcasted_iota(jnp.int32, (m1, BB), 0) // KVG
              == lax.broadcasted_iota(jnp.int32, (m1, BB), 1)).astype(jnp.float32)
        e2 = (lax.broadcasted_iota(jnp.int32, (m2, BB), 0) // KQG
              == lax.broadcasted_iota(jnp.int32, (m2, BB), 1)).astype(jnp.float32)
        vg = vg_ref[...]
        qg = qg_ref[...]
        q = q_ref[...]

        def f32dot(a, b):
            return jnp.dot(a, b, preferred_element_type=jnp.float32)

        y1 = f32dot(vg, ws1t_ref[...]) + f32dot(e1, f32dot(q, ws1b_ref[...]))
        z1 = f32dot(vg, wn1t_ref[...]) + f32dot(e1, f32dot(q, wn1b_ref[...]))
        h1c = jnp.maximum(y1 + f32dot(a1_ref[...], z1), 0.0)
        y2 = f32dot(qg, ws2t_ref[...]) + f32dot(e2, f32dot(q, ws2b_ref[...]))
        z2 = f32dot(qg, wn2t_ref[...]) + f32dot(e2, f32dot(q, wn2b_ref[...]))
        h2c = jnp.maximum(y2 + f32dot(a2_ref[...], z2), 0.0)
        h1s_ref[c] = h1c
        h2s_ref[c] = h2c
        ps12 = lax.dot_general(h1c, h2c, dn, preferred_element_type=jnp.float32)
        ps21 = lax.dot_general(h2c, h1c, dn, preferred_element_type=jnp.float32)

        @pl.when(c == 0)
        def _():
            s12_ref[...] = ps12
            s21_ref[...] = ps21

        @pl.when(c > 0)
        def _():
            s12_ref[...] += ps12
            s21_ref[...] += ps21

        @pl.when(c == NCT - 1)
        def _():
            scale = 1.0 / (DGM ** 0.5)
            s12_ref[...] = _softmax_last(s12_ref[...] * scale + m12_ref[...])
            s21_ref[...] = _softmax_last(s21_ref[...] * scale + m21_ref[...])

    @pl.when(c == NCT)
    def _():
        a12 = s12_ref[...]
        a21 = s21_ref[...]
        for cc in range(NCT):
            h1c = h1s_ref[cc]
            h2c = h2s_ref[cc]
            o1_ref[:, cc * BNL:(cc + 1) * BNL] = h1c + jnp.dot(
                a12, h2c, preferred_element_type=jnp.float32)
            o2_ref[:, cc * BNL:(cc + 1) * BNL] = h2c + jnp.dot(
                a21, h1c, preferred_element_type=jnp.float32)


def _layer_a(vg2d, qg_enc, qenc, a1, a2, m12, m21, ws1, wn1, ws2, wn2):
    m1 = BB * KVG
    m2 = BB * KQG

    def const(shape):
        return pl.BlockSpec(shape, lambda c: (0, 0))

    def wtop(shape):
        return pl.BlockSpec(shape, lambda c: (0, jnp.minimum(c, NCT - 1)))

    def wbot(shape):
        return pl.BlockSpec(shape, lambda c: (1, jnp.minimum(c, NCT - 1)))

    wt = (DGM, BNL)
    return pl.pallas_call(
        _layer_a_body,
        grid=(NCT + 1,),
        in_specs=[
            const((m1, DGM)), const((m2, DGM)), const((BB, DGM)),
            const((m1, m1)), const((m2, m2)),
            const((m1, m2)), const((m2, m1)),
            wtop(wt), wbot(wt), wtop(wt), wbot(wt),
            wtop(wt), wbot(wt), wtop(wt), wbot(wt),
        ],
        out_specs=[
            pl.BlockSpec((m1, DGM), lambda c: (0, 0)),
            pl.BlockSpec((m2, DGM), lambda c: (0, 0)),
        ],
        out_shape=[
            jax.ShapeDtypeStruct((m1, DGM), jnp.float32),
            jax.ShapeDtypeStruct((m2, DGM), jnp.float32),
        ],
        scratch_shapes=[
            pltpu.VMEM((NCT, m1, BNL), jnp.float32),
            pltpu.VMEM((NCT, m2, BNL), jnp.float32),
            pltpu.VMEM((m1, m2), jnp.float32),
            pltpu.VMEM((m2, m1), jnp.float32),
        ],
    )(vg2d, qg_enc, qenc, a1, a2, m12, m21,
      ws1, ws1, wn1, wn1, ws2, ws2, wn2, wn2)


def _layer(x1, x2, a1, a2, m12, m21, ws1, wn1, ws2, wn2):
    m1, d = x1.shape
    m2 = x2.shape[0]

    def const(shape):
        return pl.BlockSpec(shape, lambda c: (0, 0))

    def wspec(shape):
        return pl.BlockSpec(shape, lambda c: (0, jnp.minimum(c, NCT - 1)))

    return pl.pallas_call(
        _layer_body,
        grid=(NCT + 1,),
        in_specs=[
            const((m1, d)), const((m2, d)),
            const((m1, m1)), const((m2, m2)),
            const((m1, m2)), const((m2, m1)),
            wspec((d, BNL)), wspec((d, BNL)),
            wspec((d, BNL)), wspec((d, BNL)),
        ],
        out_specs=[
            pl.BlockSpec((m1, DGM), lambda c: (0, 0)),
            pl.BlockSpec((m2, DGM), lambda c: (0, 0)),
        ],
        out_shape=[
            jax.ShapeDtypeStruct((m1, DGM), jnp.float32),
            jax.ShapeDtypeStruct((m2, DGM), jnp.float32),
        ],
        scratch_shapes=[
            pltpu.VMEM((NCT, m1, BNL), jnp.float32),
            pltpu.VMEM((NCT, m2, BNL), jnp.float32),
            pltpu.VMEM((m1, m2), jnp.float32),
            pltpu.VMEM((m2, m1), jnp.float32),
        ],
    )(x1, x2, a1, a2, m12, m21, ws1, wn1, ws2, wn2)


def _head1_body(x2_ref, q_ref, w1_ref, b1_ref, o_ref):
    ffs = [jnp.max(x2_ref[b * KQG:(b + 1) * KQG, :], axis=0, keepdims=True)
           for b in range(BB)]
    ff = jnp.concatenate(ffs, axis=0)
    h = jnp.maximum(q_ref[...], 0.0) * ff
    o_ref[...] = jnp.maximum(
        jnp.dot(h, w1_ref[...], preferred_element_type=jnp.float32) + b1_ref[...],
        0.0)


def _head1(x2, qenc, w1, b1):
    bn = 512
    g = (OUTD + bn - 1) // bn
    return pl.pallas_call(
        _head1_body,
        grid=(g,),
        in_specs=[
            pl.BlockSpec((BB * KQG, DGM), lambda c: (0, 0)),
            pl.BlockSpec((BB, DGM), lambda c: (0, 0)),
            pl.BlockSpec((DGM, bn), lambda c: (0, c)),
            pl.BlockSpec((1, bn), lambda c: (0, c)),
        ],
        out_specs=pl.BlockSpec((BB, bn), lambda c: (0, c)),
        out_shape=jax.ShapeDtypeStruct((BB, OUTD), jnp.float32),
    )(x2, qenc, w1, b1)


def _head2_body(h_ref, w2_ref, b2_ref, o_ref):
    o_ref[...] = jnp.dot(h_ref[...], w2_ref[...],
                         preferred_element_type=jnp.float32) + b2_ref[...]


def _head2(hid1, w2, b2):
    bn = 512
    g = (OUTD + bn - 1) // bn
    return pl.pallas_call(
        _head2_body,
        grid=(g,),
        in_specs=[
            pl.BlockSpec((BB, OUTD), lambda c: (0, 0)),
            pl.BlockSpec((OUTD, bn), lambda c: (0, c)),
            pl.BlockSpec((1, bn), lambda c: (0, c)),
        ],
        out_specs=pl.BlockSpec((BB, bn), lambda c: (0, c)),
        out_shape=jax.ShapeDtypeStruct((BB, OUTD), jnp.float32),
    )(hid1, w2, b2)


def kernel(question, vg_nodes, vg_edges, qg_nodes, qg_edges, qglen, qlen,
           emb_table, Wf, Uf, bif, bhf, Wb, Ub, bib, bhb,
           Ws1a, Wn1a, Ws2a, Wn2a, Ws1b, Wn1b, Ws2b, Wn2b, W1, b1, W2, b2):
    tokq = question.astype(jnp.int32)
    tokg = qg_nodes.reshape(BB * KQG, NWORD).astype(jnp.int32)
    tokg = jnp.pad(tokg, ((0, 0), (0, QL - NWORD)))
    tok_f = jnp.concatenate([tokq, tokg], axis=0)
    lens = jnp.concatenate([qlen.astype(jnp.int32),
                            qglen.reshape(-1).astype(jnp.int32)])
    tt = jnp.arange(QL, dtype=jnp.int32)
    pos = jnp.clip(lens[:, None] - 1 - tt[None, :], 0, QL - 1)
    tok_r = jnp.take_along_axis(tok_f, pos, axis=1)
    ids = jnp.concatenate([
        tok_f.T.reshape(-1), tok_r.T.reshape(-1),
        jnp.zeros((NIDS_PAD - NIDS,), jnp.int32)]).astype(jnp.int32)
    tpad = _pad_table(emb_table)
    G = _sc_gather(tpad, ids)

    H = _gru_call(G, lens.reshape(SEQ, 1),
                  Wf, Uf, bif.reshape(1, -1), bhf.reshape(1, -1),
                  Wb, Ub, bib.reshape(1, -1), bhb.reshape(1, -1))
    qenc = H[:BB]
    qg_enc = H[BB:]

    roff1 = (jnp.arange(BB * KVG, dtype=jnp.int32) // KVG * KVG)[:, None]
    ge1 = vg_edges.reshape(BB * KVG, NBR).astype(jnp.int32) + roff1
    roff2 = (jnp.arange(BB * KQG, dtype=jnp.int32) // KQG * KQG)[:, None]
    ge2 = qg_edges.reshape(BB * KQG, NBR).astype(jnp.int32) + roff2
    A1, A2, M12, M21 = _adj_call(ge1, ge2, vg_nodes, qg_nodes.astype(jnp.int32))

    x1, x2 = _layer_a(vg_nodes.reshape(BB * KVG, DVG), qg_enc, qenc,
                      A1, A2, M12, M21, Ws1a, Wn1a, Ws2a, Wn2a)
    x1, x2 = _layer(x1, x2, A1, A2, M12, M21, Ws1b, Wn1b, Ws2b, Wn2b)

    hid1 = _head1(x2, qenc, W1, b1.reshape(1, OUTD))
    return _head2(hid1, W2, b2.reshape(1, OUTD))
